# baseline (speedup 1.0000x reference)
import dataclasses

import jax
import jax.numpy as jnp
from jax import lax
from jax.experimental import pallas as pl
from jax.experimental.pallas import tpu as pltpu
from jax.experimental.pallas import tpu_sc as plsc

B, N, D = 16, 512, 128
E = 8
H1, H2 = 256, 256
NA = B * N
T = 256
P = NA + E * T
NT = P // T
SENT = NA
NW = 32
NS = 16
SLICE = P // NW
GCH = 80
CHUNK = NA // NS


def _route_body(z_ref, dest_ref, teid_ref):
    z = z_ref[...]
    ri = lax.broadcasted_iota(jnp.int32, (128, 128), 0)
    ci = lax.broadcasted_iota(jnp.int32, (128, 128), 1)
    upper_incl = (ri <= ci).astype(jnp.float32)
    ones128 = jnp.ones((128, 128), jnp.float32)
    r64 = lax.broadcasted_iota(jnp.int32, (64, 64), 0)
    c64 = lax.broadcasted_iota(jnp.int32, (64, 64), 1)
    lower_strict = (c64 < r64).astype(jnp.float32)
    destf = jnp.zeros((64, 128), jnp.float32)
    pstart = jnp.int32(0)
    pends = []
    for e in range(E):
        m = z == e
        mf = m.astype(jnp.float32)
        rowcum = lax.dot(mf, upper_incl)
        rowsum = lax.dot(mf, ones128)
        prefrows = lax.dot(lower_strict, rowsum)
        rank = rowcum + prefrows - mf
        c = jnp.sum(m.astype(jnp.int32))
        pc = ((c + T - 1) // T) * T
        destf = destf + mf * (rank + pstart.astype(jnp.float32))
        pstart = pstart + pc
        pends.append(pstart)
    dest_ref[...] = destf.astype(jnp.int32)
    tv = lax.broadcasted_iota(jnp.int32, (8, 128), 1) * T
    acc = jnp.zeros((8, 128), jnp.int32)
    for pe in pends:
        acc = acc + (tv >= pe).astype(jnp.int32)
    teid_ref[...] = jnp.minimum(acc, 7)


def _route(z2d):
    return pl.pallas_call(
        _route_body,
        out_shape=(
            jax.ShapeDtypeStruct((64, 128), jnp.int32),
            jax.ShapeDtypeStruct((8, 128), jnp.int32),
        ),
    )(z2d)


def _sc_body(dest_hbm, x_hbm, xg_hbm, bid_hbm,
             dest_v, bid_v, initb_v, xchunk_v, bidout_v,
             xg_sh, bid_sh, sem):
    sid = lax.axis_index("s")
    wid = sid * 2 + lax.axis_index("c")
    base = wid * SLICE

    splat16 = jnp.full((16,), 16, jnp.int32)

    @pl.loop(0, P // NS, step=16)
    def _init(i):
        initb_v[pl.ds(i, 16)] = splat16

    pltpu.sync_copy(initb_v, bid_sh.at[pl.ds(sid * (P // NS), P // NS)])

    pltpu.sync_copy(dest_hbm.at[pl.ds(sid * 4, 4)], dest_v)
    mybid = jnp.full((16,), 0, jnp.int32) + sid

    @pl.loop(0, 4)
    def _bidfill(j):
        @pl.loop(0, 128, step=16)
        def _(i):
            bid_v[j, pl.ds(i, 16)] = mybid

    plsc.subcore_barrier()

    for j in range(4):
        pltpu.sync_copy(x_hbm.at[pl.ds(sid * CHUNK + j * 128, 128)], xchunk_v)
        pltpu.sync_copy(xchunk_v, xg_sh.at[dest_v.at[j]])
        pltpu.sync_copy(bid_v.at[j], bid_sh.at[dest_v.at[j]])

    plsc.subcore_barrier()

    pltpu.sync_copy(xg_sh.at[pl.ds(base, SLICE)], xg_hbm.at[pl.ds(base, SLICE)])
    pltpu.sync_copy(bid_sh.at[pl.ds(base, SLICE)], bidout_v)
    pltpu.sync_copy(bidout_v, bid_hbm.at[pl.ds(base, SLICE)])


def _sc_gather(dest2d, x_flat):
    mesh = plsc.VectorSubcoreMesh(
        core_axis_name="c", subcore_axis_name="s", num_cores=2, num_subcores=16
    )
    cp = pltpu.CompilerParams()
    if "needs_layout_passes" in pltpu.CompilerParams.__dataclass_fields__:
        cp = dataclasses.replace(cp, needs_layout_passes=False)
    run = pl.kernel(
        _sc_body,
        out_type=(
            jax.ShapeDtypeStruct((P, D), jnp.float32),
            jax.ShapeDtypeStruct((P,), jnp.int32),
        ),
        mesh=mesh,
        compiler_params=cp,
        scratch_types=[
            pltpu.VMEM((4, 128), jnp.int32),
            pltpu.VMEM((4, 128), jnp.int32),
            pltpu.VMEM((P // NS,), jnp.int32),
            pltpu.VMEM((128, D), jnp.float32),
            pltpu.VMEM((SLICE,), jnp.int32),
            pltpu.VMEM_SHARED((P, D), jnp.float32),
            pltpu.VMEM_SHARED((P,), jnp.int32),
            pltpu.SemaphoreType.DMA,
        ],
    )
    return run(dest2d, x_flat)


NI = 8


def _mlp_body(s_ref, xg_ref, bid_ref, *refs):
    out_ref = refs[-1]
    wrefs = tuple(
        (refs[3 * h], refs[3 * h + 1], refs[3 * h + 2]) for h in range(NI))
    t = pl.program_id(0)
    contrib = jnp.zeros((1, 16), jnp.float32)
    for half in range(NI):
        w1_ref, w2_ref, aux_ref = wrefs[half]
        aux = aux_ref[0]
        sl = pl.ds(half * T, T)
        bid = bid_ref[sl, :]
        x = jnp.where(bid < 16, xg_ref[sl, :], 0.0)
        oh = (bid == lax.broadcasted_iota(jnp.int32, (1, 16), 1)).astype(
            jnp.float32)
        h = jnp.tanh(lax.dot(x, w1_ref[0]) + aux[0:1, :])
        h = jnp.tanh(lax.dot(h, w2_ref[0]) + aux[1:2, :])
        y = jnp.sum(h * aux[2:3, :], axis=1, keepdims=True)
        y = y + aux[3:4, 0:1]
        contrib = contrib + jnp.sum(oh * y, axis=0, keepdims=True)

    @pl.when(t == 0)
    def _():
        out_ref[...] = jnp.zeros((8, 16), jnp.float32)

    out_ref[...] = out_ref[...] + jnp.broadcast_to(contrib, (8, 16))


def _mlp(teid, xg, bidcol, W1, W2, aux):
    wspecs = []
    for half in range(NI):
        wspecs.append(pl.BlockSpec(
            (1, D, H1), lambda t, s, h=half: (s[NI * t + h], 0, 0)))
        wspecs.append(pl.BlockSpec(
            (1, H1, H2), lambda t, s, h=half: (s[NI * t + h], 0, 0)))
        wspecs.append(pl.BlockSpec(
            (1, 8, H2), lambda t, s, h=half: (s[NI * t + h], 0, 0)))
    grid_spec = pltpu.PrefetchScalarGridSpec(
        num_scalar_prefetch=1,
        grid=(NT // NI,),
        in_specs=[
            pl.BlockSpec((NI * T, D), lambda t, s: (t, 0)),
            pl.BlockSpec((NI * T, 1), lambda t, s: (t, 0)),
            *wspecs,
        ],
        out_specs=pl.BlockSpec((8, 16), lambda t, s: (0, 0)),
    )
    return pl.pallas_call(
        _mlp_body,
        grid_spec=grid_spec,
        out_shape=jax.ShapeDtypeStruct((8, 16), jnp.float32),
    )(teid, xg, bidcol, *([W1, W2, aux] * NI))


def kernel(X, Z, W1, b1, W2, b2, W3, b3):
    x_flat = X.reshape(NA, D)
    z2d = Z.reshape(64, 128).astype(jnp.int32)
    dest2d, teid2d = _route(z2d)
    xg, bid = _sc_gather(dest2d, x_flat)
    teid = teid2d[0, :NT]
    aux = jnp.concatenate(
        [
            b1[:, None, :],
            b2[:, None, :],
            W3.reshape(E, 1, H2),
            jnp.broadcast_to(b3.reshape(E, 1, 1), (E, 1, H2)),
            jnp.zeros((E, 4, H2), jnp.float32),
        ],
        axis=1,
    )
    out = _mlp(teid, xg, bid.reshape(P, 1), W1, W2, aux)
    return out[0]


# speedup vs baseline: 1.0828x; 1.0828x over previous
import dataclasses

import jax
import jax.numpy as jnp
from jax import lax
from jax.experimental import pallas as pl
from jax.experimental.pallas import tpu as pltpu
from jax.experimental.pallas import tpu_sc as plsc

B, N, D = 16, 512, 128
E = 8
H1, H2 = 256, 256
NA = B * N
T = 256
P = NA + E * T
NT = P // T
SENT = NA
NW = 32
NS = 16
SLICE = P // NW
GCH = 80
CHUNK = NA // NS


def _route_body(z_ref, dest_ref, teid_ref):
    z = z_ref[...]
    ri = lax.broadcasted_iota(jnp.int32, (128, 128), 0)
    ci = lax.broadcasted_iota(jnp.int32, (128, 128), 1)
    upper_incl = (ri <= ci).astype(jnp.float32)
    ones128 = jnp.ones((128, 128), jnp.float32)
    r64 = lax.broadcasted_iota(jnp.int32, (64, 64), 0)
    c64 = lax.broadcasted_iota(jnp.int32, (64, 64), 1)
    lower_strict = (c64 < r64).astype(jnp.float32)
    destf = jnp.zeros((64, 128), jnp.float32)
    pstart = jnp.int32(0)
    pends = []
    for e in range(E):
        m = z == e
        mf = m.astype(jnp.float32)
        rowcum = lax.dot(mf, upper_incl)
        rowsum = lax.dot(mf, ones128)
        prefrows = lax.dot(lower_strict, rowsum)
        rank = rowcum + prefrows - mf
        c = jnp.sum(m.astype(jnp.int32))
        pc = ((c + T - 1) // T) * T
        destf = destf + mf * (rank + pstart.astype(jnp.float32))
        pstart = pstart + pc
        pends.append(pstart)
    dest_ref[...] = destf.astype(jnp.int32)
    tv = lax.broadcasted_iota(jnp.int32, (8, 128), 1) * T
    acc = jnp.zeros((8, 128), jnp.int32)
    for pe in pends:
        acc = acc + (tv >= pe).astype(jnp.int32)
    teid_ref[...] = jnp.minimum(acc, 7)


def _route(z2d):
    return pl.pallas_call(
        _route_body,
        out_shape=(
            jax.ShapeDtypeStruct((64, 128), jnp.int32),
            jax.ShapeDtypeStruct((8, 128), jnp.int32),
        ),
    )(z2d)


def _sc_body(dest_hbm, x_hbm, xg_hbm, bid_hbm,
             dest_v, bid_v, initb_v, xchunk_v, xchunk2_v, bidout_v,
             xg_sh, bid_sh, sem, sem2):
    sid = lax.axis_index("s")
    wid = sid * 2 + lax.axis_index("c")
    base = wid * SLICE

    splat16 = jnp.full((16,), 16, jnp.int32)

    @pl.loop(0, P // NS, step=16)
    def _init(i):
        initb_v[pl.ds(i, 16)] = splat16

    pltpu.sync_copy(initb_v, bid_sh.at[pl.ds(sid * (P // NS), P // NS)])

    pltpu.sync_copy(dest_hbm.at[pl.ds(sid * 4, 4)], dest_v)
    mybid = jnp.full((16,), 0, jnp.int32) + sid

    @pl.loop(0, 4)
    def _bidfill(j):
        @pl.loop(0, 128, step=16)
        def _(i):
            bid_v[j, pl.ds(i, 16)] = mybid

    plsc.subcore_barrier()

    xbufs = (xchunk_v, xchunk2_v)
    pend = []
    waited = set()
    for j in range(4):
        buf = xbufs[j % 2]
        if j >= 2:
            pend[2 * (j - 2)].wait()
            waited.add(2 * (j - 2))
        pltpu.sync_copy(x_hbm.at[pl.ds(sid * CHUNK + j * 128, 128)], buf)
        pend.append(pltpu.async_copy(buf, xg_sh.at[dest_v.at[j]], sem))
        pend.append(pltpu.async_copy(bid_v.at[j], bid_sh.at[dest_v.at[j]],
                                     sem2))
    for k, cp in enumerate(pend):
        if k not in waited:
            cp.wait()

    plsc.subcore_barrier()

    pltpu.sync_copy(xg_sh.at[pl.ds(base, SLICE)], xg_hbm.at[pl.ds(base, SLICE)])
    pltpu.sync_copy(bid_sh.at[pl.ds(base, SLICE)], bidout_v)
    pltpu.sync_copy(bidout_v, bid_hbm.at[pl.ds(base, SLICE)])


def _sc_gather(dest2d, x_flat):
    mesh = plsc.VectorSubcoreMesh(
        core_axis_name="c", subcore_axis_name="s", num_cores=2, num_subcores=16
    )
    cp = pltpu.CompilerParams()
    if "needs_layout_passes" in pltpu.CompilerParams.__dataclass_fields__:
        cp = dataclasses.replace(cp, needs_layout_passes=False)
    run = pl.kernel(
        _sc_body,
        out_type=(
            jax.ShapeDtypeStruct((P, D), jnp.float32),
            jax.ShapeDtypeStruct((P,), jnp.int32),
        ),
        mesh=mesh,
        compiler_params=cp,
        scratch_types=[
            pltpu.VMEM((4, 128), jnp.int32),
            pltpu.VMEM((4, 128), jnp.int32),
            pltpu.VMEM((P // NS,), jnp.int32),
            pltpu.VMEM((128, D), jnp.float32),
            pltpu.VMEM((128, D), jnp.float32),
            pltpu.VMEM((SLICE,), jnp.int32),
            pltpu.VMEM_SHARED((P, D), jnp.float32),
            pltpu.VMEM_SHARED((P,), jnp.int32),
            pltpu.SemaphoreType.DMA,
            pltpu.SemaphoreType.DMA,
        ],
    )
    return run(dest2d, x_flat)


NI = 4


def _mlp_body(s_ref, xg_ref, bid_ref, *refs):
    out_ref = refs[-1]
    wrefs = tuple(
        (refs[3 * h], refs[3 * h + 1], refs[3 * h + 2]) for h in range(NI))
    t = pl.program_id(0)
    contrib = jnp.zeros((1, 16), jnp.float32)
    for half in range(NI):
        w1_ref, w2_ref, aux_ref = wrefs[half]
        aux = aux_ref[0]
        sl = pl.ds(half * T, T)
        bid = bid_ref[sl, :]
        x = jnp.where(bid < 16, xg_ref[sl, :], 0.0)
        oh = (bid == lax.broadcasted_iota(jnp.int32, (1, 16), 1)).astype(
            jnp.float32)
        h = jnp.tanh(lax.dot(x, w1_ref[0]) + aux[0:1, :])
        h = jnp.tanh(lax.dot(h, w2_ref[0]) + aux[1:2, :])
        y = jnp.sum(h * aux[2:3, :], axis=1, keepdims=True)
        y = y + aux[3:4, 0:1]
        contrib = contrib + jnp.sum(oh * y, axis=0, keepdims=True)

    @pl.when(t == 0)
    def _():
        out_ref[...] = jnp.zeros((8, 16), jnp.float32)

    out_ref[...] = out_ref[...] + jnp.broadcast_to(contrib, (8, 16))


def _mlp(teid, xg, bidcol, W1, W2, aux):
    wspecs = []
    for half in range(NI):
        wspecs.append(pl.BlockSpec(
            (1, D, H1), lambda t, s, h=half: (s[NI * t + h], 0, 0)))
        wspecs.append(pl.BlockSpec(
            (1, H1, H2), lambda t, s, h=half: (s[NI * t + h], 0, 0)))
        wspecs.append(pl.BlockSpec(
            (1, 8, H2), lambda t, s, h=half: (s[NI * t + h], 0, 0)))
    grid_spec = pltpu.PrefetchScalarGridSpec(
        num_scalar_prefetch=1,
        grid=(NT // NI,),
        in_specs=[
            pl.BlockSpec((NI * T, D), lambda t, s: (t, 0)),
            pl.BlockSpec((NI * T, 1), lambda t, s: (t, 0)),
            *wspecs,
        ],
        out_specs=pl.BlockSpec((8, 16), lambda t, s: (0, 0)),
    )
    return pl.pallas_call(
        _mlp_body,
        grid_spec=grid_spec,
        out_shape=jax.ShapeDtypeStruct((8, 16), jnp.float32),
    )(teid, xg, bidcol, *([W1, W2, aux] * NI))


def kernel(X, Z, W1, b1, W2, b2, W3, b3):
    x_flat = X.reshape(NA, D)
    z2d = Z.reshape(64, 128).astype(jnp.int32)
    dest2d, teid2d = _route(z2d)
    xg, bid = _sc_gather(dest2d, x_flat)
    teid = teid2d[0, :NT]
    aux = jnp.concatenate(
        [
            b1[:, None, :],
            b2[:, None, :],
            W3.reshape(E, 1, H2),
            jnp.broadcast_to(b3.reshape(E, 1, 1), (E, 1, H2)),
            jnp.zeros((E, 4, H2), jnp.float32),
        ],
        axis=1,
    )
    out = _mlp(teid, xg, bid.reshape(P, 1), W1, W2, aux)
    return out[0]


# speedup vs baseline: 1.1033x; 1.0189x over previous
import dataclasses

import jax
import jax.numpy as jnp
from jax import lax
from jax.experimental import pallas as pl
from jax.experimental.pallas import tpu as pltpu
from jax.experimental.pallas import tpu_sc as plsc

B, N, D = 16, 512, 128
E = 8
H1, H2 = 256, 256
NA = B * N
T = 128
P = NA + E * T
NT = P // T
SENT = NA
NW = 32
NS = 16
SLICE = P // NW
GCH = 80
CHUNK = NA // NS


def _route_body(z_ref, dest_ref, teid_ref):
    z = z_ref[...]
    ri = lax.broadcasted_iota(jnp.int32, (128, 128), 0)
    ci = lax.broadcasted_iota(jnp.int32, (128, 128), 1)
    upper_incl = (ri <= ci).astype(jnp.float32)
    ones128 = jnp.ones((128, 128), jnp.float32)
    r64 = lax.broadcasted_iota(jnp.int32, (64, 64), 0)
    c64 = lax.broadcasted_iota(jnp.int32, (64, 64), 1)
    lower_strict = (c64 < r64).astype(jnp.float32)
    destf = jnp.zeros((64, 128), jnp.float32)
    pstart = jnp.int32(0)
    pends = []
    for e in range(E):
        m = z == e
        mf = m.astype(jnp.float32)
        rowcum = lax.dot(mf, upper_incl)
        rowsum = lax.dot(mf, ones128)
        prefrows = lax.dot(lower_strict, rowsum)
        rank = rowcum + prefrows - mf
        c = jnp.sum(m.astype(jnp.int32))
        pc = ((c + T - 1) // T) * T
        destf = destf + mf * (rank + pstart.astype(jnp.float32))
        pstart = pstart + pc
        pends.append(pstart)
    dest_ref[...] = destf.astype(jnp.int32)
    tv = lax.broadcasted_iota(jnp.int32, (8, 128), 1) * T
    acc = jnp.zeros((8, 128), jnp.int32)
    for pe in pends:
        acc = acc + (tv >= pe).astype(jnp.int32)
    teid_ref[...] = jnp.minimum(acc, 7)


def _route(z2d):
    return pl.pallas_call(
        _route_body,
        out_shape=(
            jax.ShapeDtypeStruct((64, 128), jnp.int32),
            jax.ShapeDtypeStruct((8, 128), jnp.int32),
        ),
    )(z2d)


def _sc_body(dest_hbm, x_hbm, xg_hbm, bid_hbm,
             dest_v, bid_v, initb_v, xchunk_v, xchunk2_v, bidout_v,
             xg_sh, bid_sh, sem, sem2):
    sid = lax.axis_index("s")
    wid = sid * 2 + lax.axis_index("c")
    base = wid * SLICE

    splat16 = jnp.full((16,), 16, jnp.int32)

    @pl.loop(0, P // NS, step=16)
    def _init(i):
        initb_v[pl.ds(i, 16)] = splat16

    pltpu.sync_copy(initb_v, bid_sh.at[pl.ds(sid * (P // NS), P // NS)])

    pltpu.sync_copy(dest_hbm.at[pl.ds(sid * 4, 4)], dest_v)
    mybid = jnp.full((16,), 0, jnp.int32) + sid

    @pl.loop(0, 4)
    def _bidfill(j):
        @pl.loop(0, 128, step=16)
        def _(i):
            bid_v[j, pl.ds(i, 16)] = mybid

    plsc.subcore_barrier()

    xbufs = (xchunk_v, xchunk2_v)
    pend = []
    waited = set()
    for j in range(4):
        buf = xbufs[j % 2]
        if j >= 2:
            pend[2 * (j - 2)].wait()
            waited.add(2 * (j - 2))
        pltpu.sync_copy(x_hbm.at[pl.ds(sid * CHUNK + j * 128, 128)], buf)
        pend.append(pltpu.async_copy(buf, xg_sh.at[dest_v.at[j]], sem))
        pend.append(pltpu.async_copy(bid_v.at[j], bid_sh.at[dest_v.at[j]],
                                     sem2))
    for k, cp in enumerate(pend):
        if k not in waited:
            cp.wait()

    plsc.subcore_barrier()

    pltpu.sync_copy(xg_sh.at[pl.ds(base, SLICE)], xg_hbm.at[pl.ds(base, SLICE)])
    pltpu.sync_copy(bid_sh.at[pl.ds(base, SLICE)], bidout_v)
    pltpu.sync_copy(bidout_v, bid_hbm.at[pl.ds(base, SLICE)])


def _sc_gather(dest2d, x_flat):
    mesh = plsc.VectorSubcoreMesh(
        core_axis_name="c", subcore_axis_name="s", num_cores=2, num_subcores=16
    )
    cp = pltpu.CompilerParams()
    if "needs_layout_passes" in pltpu.CompilerParams.__dataclass_fields__:
        cp = dataclasses.replace(cp, needs_layout_passes=False)
    run = pl.kernel(
        _sc_body,
        out_type=(
            jax.ShapeDtypeStruct((P, D), jnp.float32),
            jax.ShapeDtypeStruct((P,), jnp.int32),
        ),
        mesh=mesh,
        compiler_params=cp,
        scratch_types=[
            pltpu.VMEM((4, 128), jnp.int32),
            pltpu.VMEM((4, 128), jnp.int32),
            pltpu.VMEM((P // NS,), jnp.int32),
            pltpu.VMEM((128, D), jnp.float32),
            pltpu.VMEM((128, D), jnp.float32),
            pltpu.VMEM((SLICE,), jnp.int32),
            pltpu.VMEM_SHARED((P, D), jnp.float32),
            pltpu.VMEM_SHARED((P,), jnp.int32),
            pltpu.SemaphoreType.DMA,
            pltpu.SemaphoreType.DMA,
        ],
    )
    return run(dest2d, x_flat)


NI = 8


def _mlp_body(s_ref, xg_ref, bid_ref, *refs):
    out_ref = refs[-1]
    wrefs = tuple(
        (refs[3 * h], refs[3 * h + 1], refs[3 * h + 2]) for h in range(NI))
    t = pl.program_id(0)
    contrib = jnp.zeros((1, 16), jnp.float32)
    for half in range(NI):
        w1_ref, w2_ref, aux_ref = wrefs[half]
        aux = aux_ref[0]
        sl = pl.ds(half * T, T)
        bid = bid_ref[sl, :]
        x = jnp.where(bid < 16, xg_ref[sl, :], 0.0)
        oh = (bid == lax.broadcasted_iota(jnp.int32, (1, 16), 1)).astype(
            jnp.float32)
        h = jnp.tanh(lax.dot(x, w1_ref[0]) + aux[0:1, :])
        h = jnp.tanh(lax.dot(h, w2_ref[0]) + aux[1:2, :])
        y = jnp.sum(h * aux[2:3, :], axis=1, keepdims=True)
        y = y + aux[3:4, 0:1]
        contrib = contrib + jnp.sum(oh * y, axis=0, keepdims=True)

    @pl.when(t == 0)
    def _():
        out_ref[...] = jnp.zeros((8, 16), jnp.float32)

    out_ref[...] = out_ref[...] + jnp.broadcast_to(contrib, (8, 16))


def _mlp(teid, xg, bidcol, W1, W2, aux):
    wspecs = []
    for half in range(NI):
        wspecs.append(pl.BlockSpec(
            (1, D, H1), lambda t, s, h=half: (s[NI * t + h], 0, 0)))
        wspecs.append(pl.BlockSpec(
            (1, H1, H2), lambda t, s, h=half: (s[NI * t + h], 0, 0)))
        wspecs.append(pl.BlockSpec(
            (1, 8, H2), lambda t, s, h=half: (s[NI * t + h], 0, 0)))
    grid_spec = pltpu.PrefetchScalarGridSpec(
        num_scalar_prefetch=1,
        grid=(NT // NI,),
        in_specs=[
            pl.BlockSpec((NI * T, D), lambda t, s: (t, 0)),
            pl.BlockSpec((NI * T, 1), lambda t, s: (t, 0)),
            *wspecs,
        ],
        out_specs=pl.BlockSpec((8, 16), lambda t, s: (0, 0)),
    )
    return pl.pallas_call(
        _mlp_body,
        grid_spec=grid_spec,
        out_shape=jax.ShapeDtypeStruct((8, 16), jnp.float32),
    )(teid, xg, bidcol, *([W1, W2, aux] * NI))


def kernel(X, Z, W1, b1, W2, b2, W3, b3):
    x_flat = X.reshape(NA, D)
    z2d = Z.reshape(64, 128).astype(jnp.int32)
    dest2d, teid2d = _route(z2d)
    xg, bid = _sc_gather(dest2d, x_flat)
    teid = teid2d[0, :NT]
    aux = jnp.concatenate(
        [
            b1[:, None, :],
            b2[:, None, :],
            W3.reshape(E, 1, H2),
            jnp.broadcast_to(b3.reshape(E, 1, 1), (E, 1, H2)),
            jnp.zeros((E, 4, H2), jnp.float32),
        ],
        axis=1,
    )
    out = _mlp(teid, xg, bid.reshape(P, 1), W1, W2, aux)
    return out[0]
